# Initial kernel scaffold; baseline (speedup 1.0000x reference)
import jax
import jax.numpy as jnp
from jax.experimental import pallas as pl


def kernel(x_user, x_item, edge_index_u2i, edge_index_i2u, Wl0_u2i, bl0_u2i, Wr0_u2i, Wl0_i2u, bl0_i2u, Wr0_i2u, Wl1_u2i, bl1_u2i, Wr1_u2i, Wl1_i2u, bl1_i2u, Wr1_i2u, gamma_user, beta_user, gamma_item, beta_item):
    raise NotImplementedError("write your pallas kernel here")


# speedup vs baseline: 5.0848x; 5.0848x over previous
import jax
import jax.numpy as jnp
from jax import lax
from jax.experimental import pallas as pl
from jax.experimental.pallas import tpu as pltpu
from jax.experimental.pallas import tpu_sc as plsc

N_USER = 10000
N_ITEM = 10000
N_ALL = N_USER + N_ITEM
E = 320000
D = 128
SLOPE = 0.01

NC = 2
NS = 16
C = 128

T_TILE = ((E + NS - 1) // NS + C - 1) // C * C
E_PAD = T_TILE * NS
NSTEP = T_TILE // C

ACC_ROWS = 10240
ROWS_TILE = ACC_ROWS // NS
NRCHUNK = ROWS_TILE // C


def _make_agg():
    def body(table, srcs, dsts, zeros, acc_out, sidx, didx, rows, acc_sh, gsem):
        cid = lax.axis_index("c")
        tid = lax.axis_index("s")
        r0 = tid * ROWS_TILE
        pltpu.sync_copy(zeros.at[pl.ds(0, C)], rows.at[0])

        def zstep(i, carry):
            pltpu.sync_copy(rows.at[0], acc_sh.at[pl.ds(r0 + i * C, C)])
            return carry

        lax.fori_loop(0, NRCHUNK, zstep, 0)
        base0 = tid * T_TILE

        def step(i, carry):
            base = base0 + i * C
            pltpu.sync_copy(srcs.at[cid, pl.ds(base, C)], sidx.at[0])
            pltpu.sync_copy(dsts.at[cid, pl.ds(base, C)], didx.at[0])
            pltpu.async_copy(table.at[sidx.at[0]], rows.at[0], gsem).wait()
            pltpu.sync_copy(rows.at[0], acc_sh.at[didx.at[0]], add=True)
            return carry

        lax.fori_loop(0, NSTEP, step, 0)
        plsc.subcore_barrier()

        def dstep(i, carry):
            pltpu.sync_copy(acc_sh.at[pl.ds(r0 + i * C, C)], rows.at[0])
            pltpu.sync_copy(rows.at[0], acc_out.at[cid, pl.ds(r0 + i * C, C)])
            return carry

        lax.fori_loop(0, NRCHUNK, dstep, 0)

    return pl.kernel(
        body,
        out_type=jax.ShapeDtypeStruct((NC, ACC_ROWS, D), jnp.float32),
        mesh=plsc.VectorSubcoreMesh(core_axis_name="c", subcore_axis_name="s"),
        scratch_types=[
            pltpu.VMEM((2, C), jnp.int32),
            pltpu.VMEM((2, C), jnp.int32),
            pltpu.VMEM((1, C, D), jnp.float32),
            pltpu.VMEM_SHARED((ACC_ROWS, D), jnp.float32),
            pltpu.SemaphoreType.DMA,
        ],
    )


def _make_cnt():
    def body(dsts, ones, acc_out, didx, ones_v, cb16, cnt_sh):
        cid = lax.axis_index("c")
        tid = lax.axis_index("s")
        r0 = tid * ROWS_TILE

        def zstep(i, carry):
            pltpu.sync_copy(ones_v.at[pl.ds(0, C)],
                            cnt_sh.at[pl.ds(r0 + i * C, C)])
            return carry

        base0 = tid * T_TILE

        def step(i, carry):
            pltpu.sync_copy(dsts.at[cid, pl.ds(base0 + i * C, C)], didx.at[0])
            pltpu.sync_copy(ones_v.at[pl.ds(0, C)],
                            cnt_sh.at[didx.at[0]], add=True)
            return carry

        def dstep(i, carry):
            pltpu.sync_copy(cnt_sh.at[pl.ds(r0 + i * C, C)], cb16)
            pltpu.sync_copy(cb16, acc_out.at[cid, pl.ds(r0 + i * C, C)])
            return carry

        pltpu.sync_copy(ones.at[pl.ds(C, C)], ones_v)
        lax.fori_loop(0, NRCHUNK, zstep, 0)
        pltpu.sync_copy(ones.at[pl.ds(0, C)], ones_v)
        lax.fori_loop(0, NSTEP, step, 0)
        plsc.subcore_barrier()
        lax.fori_loop(0, NRCHUNK, dstep, 0)

    return pl.kernel(
        body,
        out_type=jax.ShapeDtypeStruct((NC, ACC_ROWS, 16), jnp.float32),
        mesh=plsc.VectorSubcoreMesh(core_axis_name="c", subcore_axis_name="s"),
        scratch_types=[
            pltpu.VMEM((2, C), jnp.int32),
            pltpu.VMEM((C, 16), jnp.float32),
            pltpu.VMEM((C, 16), jnp.float32),
            pltpu.VMEM_SHARED((ACC_ROWS, 16), jnp.float32),
        ],
    )


def _tc_layer(acc, cnt, x_all, wl, bl, wr, gamma, beta, apply_bn: bool):
    RB = 1000
    NBU = N_USER // RB

    def body(acc_ref, cnt_ref, x_ref, wl_ref, bl_ref, wr_ref, g_ref, b_ref, o_ref):
        a = acc_ref[0]
        c = jnp.maximum(cnt_ref[0, :, 0:1], 1.0)
        mean = a / c
        h = lax.dot_general(mean, wl_ref[0], (((1,), (1,)), ((), ())),
                            preferred_element_type=jnp.float32)
        h = h + bl_ref[0]
        h = h + lax.dot_general(x_ref[...], wr_ref[0], (((1,), (1,)), ((), ())),
                                preferred_element_type=jnp.float32)
        if apply_bn:
            h = h * g_ref[0] + b_ref[0]
            h = jnp.where(h >= 0, h, h * SLOPE)
        o_ref[...] = h

    grid = (N_ALL // RB,)
    typ = lambda b: b // NBU
    in_specs = [
        pl.BlockSpec((1, RB, D), lambda b: (1 - b // NBU, b % NBU, 0)),
        pl.BlockSpec((1, RB, 16), lambda b: (1 - b // NBU, b % NBU, 0)),
        pl.BlockSpec((RB, D), lambda b: (b, 0)),
        pl.BlockSpec((1, D, D), lambda b: (typ(b), 0, 0)),
        pl.BlockSpec((1, 1, D), lambda b: (typ(b), 0, 0)),
        pl.BlockSpec((1, D, D), lambda b: (typ(b), 0, 0)),
        pl.BlockSpec((1, 1, D), lambda b: (typ(b), 0, 0)),
        pl.BlockSpec((1, 1, D), lambda b: (typ(b), 0, 0)),
    ]
    return pl.pallas_call(
        body,
        grid=grid,
        in_specs=in_specs,
        out_specs=pl.BlockSpec((RB, D), lambda b: (b, 0)),
        out_shape=jax.ShapeDtypeStruct((N_ALL, D), jnp.float32),
    )(acc, cnt, x_all, wl, bl, wr, gamma, beta)


def kernel(x_user, x_item, edge_index_u2i, edge_index_i2u,
           Wl0_u2i, bl0_u2i, Wr0_u2i, Wl0_i2u, bl0_i2u, Wr0_i2u,
           Wl1_u2i, bl1_u2i, Wr1_u2i, Wl1_i2u, bl1_i2u, Wr1_i2u,
           gamma_user, beta_user, gamma_item, beta_item):
    su, du = edge_index_u2i[0], edge_index_u2i[1]
    si, di = edge_index_i2u[0], edge_index_i2u[1]

    x_all = jnp.concatenate([x_user, x_item], axis=0)
    npad = E_PAD - E
    k = jnp.arange(npad, dtype=jnp.int32)
    pad_src = k % 9973
    pad_dst = N_USER + (k % (ACC_ROWS - N_USER))
    srcs = jnp.stack([jnp.concatenate([su, pad_src]),
                      jnp.concatenate([si + N_USER, N_USER + pad_src])])
    dsts = jnp.stack([jnp.concatenate([du, pad_dst]),
                      jnp.concatenate([di, pad_dst])])
    zeros = jnp.zeros((C, D), jnp.float32)
    ones = jnp.concatenate([jnp.ones((C, 16), jnp.float32),
                            jnp.zeros((C, 16), jnp.float32)])

    cnt = _make_cnt()(dsts, ones)
    acc0 = _make_agg()(x_all, srcs, dsts, zeros)

    wl0 = jnp.stack([Wl0_i2u, Wl0_u2i])
    bl0 = jnp.stack([bl0_i2u, bl0_u2i])[:, None, :]
    wr0 = jnp.stack([Wr0_i2u, Wr0_u2i])
    gam = jnp.stack([gamma_user, gamma_item])[:, None, :]
    bet = jnp.stack([beta_user, beta_item])[:, None, :]
    h_all = _tc_layer(acc0, cnt, x_all, wl0, bl0, wr0, gam, bet, True)

    acc1 = _make_agg()(h_all, srcs, dsts, zeros)
    wl1 = jnp.stack([Wl1_i2u, Wl1_u2i])
    bl1 = jnp.stack([bl1_i2u, bl1_u2i])[:, None, :]
    wr1 = jnp.stack([Wr1_i2u, Wr1_u2i])
    out_all = _tc_layer(acc1, cnt, h_all, wl1, bl1, wr1, gam, bet, False)
    return out_all[:N_USER], out_all[N_USER:]


# speedup vs baseline: 8.6889x; 1.7088x over previous
import jax
import jax.numpy as jnp
from jax import lax
from jax.experimental import pallas as pl
from jax.experimental.pallas import tpu as pltpu
from jax.experimental.pallas import tpu_sc as plsc

N_USER = 10000
N_ITEM = 10000
N_ALL = N_USER + N_ITEM
E = 320000
D = 128
SLOPE = 0.01

NC = 2
NS = 16
C = 128

T_TILE = ((E + NS - 1) // NS + 2 * C - 1) // (2 * C) * (2 * C)
E_PAD = T_TILE * NS
NSTEP = T_TILE // C

ACC_ROWS = 10240
ROWS_TILE = ACC_ROWS // NS
NRCHUNK = ROWS_TILE // C


def _make_agg():
    def body(table, sd, zeros, acc_out, idx, rows, acc_sh, gsem0, gsem1):
        cid = lax.axis_index("c")
        tid = lax.axis_index("s")
        r0 = tid * ROWS_TILE
        pltpu.sync_copy(zeros.at[pl.ds(0, C)], rows.at[0])

        def zstep(i, carry):
            pltpu.sync_copy(rows.at[0], acc_sh.at[pl.ds(r0 + i * C, C)])
            return carry

        lax.fori_loop(0, NRCHUNK, zstep, 0)
        blk0 = tid * NSTEP

        pltpu.sync_copy(sd.at[cid, blk0], idx.at[0])
        pltpu.async_copy(table.at[idx.at[0, 0]], rows.at[0], gsem0)

        def pair(j, carry):
            i0 = 2 * j
            pltpu.sync_copy(sd.at[cid, blk0 + i0 + 1], idx.at[1])
            pltpu.async_copy(table.at[idx.at[1, 0]], rows.at[1], gsem1)
            pltpu.make_async_copy(table.at[idx.at[0, 0]], rows.at[0],
                                  gsem0).wait()
            pltpu.sync_copy(rows.at[0], acc_sh.at[idx.at[0, 1]], add=True)
            n2 = jnp.minimum(i0 + 2, NSTEP - 1)
            pltpu.sync_copy(sd.at[cid, blk0 + n2], idx.at[0])
            pltpu.async_copy(table.at[idx.at[0, 0]], rows.at[0], gsem0)
            pltpu.make_async_copy(table.at[idx.at[1, 0]], rows.at[1],
                                  gsem1).wait()
            pltpu.sync_copy(rows.at[1], acc_sh.at[idx.at[1, 1]], add=True)
            return carry

        lax.fori_loop(0, NSTEP // 2, pair, 0)
        pltpu.make_async_copy(table.at[idx.at[0, 0]], rows.at[0], gsem0).wait()
        plsc.subcore_barrier()

        def dstep(i, carry):
            pltpu.sync_copy(acc_sh.at[pl.ds(r0 + i * C, C)], rows.at[0])
            pltpu.sync_copy(rows.at[0], acc_out.at[cid, pl.ds(r0 + i * C, C)])
            return carry

        lax.fori_loop(0, NRCHUNK, dstep, 0)

    return pl.kernel(
        body,
        out_type=jax.ShapeDtypeStruct((NC, ACC_ROWS, D), jnp.float32),
        mesh=plsc.VectorSubcoreMesh(core_axis_name="c", subcore_axis_name="s"),
        scratch_types=[
            pltpu.VMEM((2, 2, C), jnp.int32),
            pltpu.VMEM((2, C, D), jnp.float32),
            pltpu.VMEM_SHARED((ACC_ROWS, D), jnp.float32),
            pltpu.SemaphoreType.DMA,
            pltpu.SemaphoreType.DMA,
        ],
    )


def _make_cnt():
    def body(sd, ones, acc_out, didx, ones_v, cb16, cnt_sh):
        cid = lax.axis_index("c")
        tid = lax.axis_index("s")
        r0 = tid * ROWS_TILE

        def zstep(i, carry):
            pltpu.sync_copy(ones_v.at[pl.ds(0, C)],
                            cnt_sh.at[pl.ds(r0 + i * C, C)])
            return carry

        blk0 = tid * NSTEP

        def step(i, carry):
            pltpu.sync_copy(sd.at[cid, blk0 + i, 1], didx.at[0])
            pltpu.sync_copy(ones_v.at[pl.ds(0, C)],
                            cnt_sh.at[didx.at[0]], add=True)
            return carry

        def dstep(i, carry):
            pltpu.sync_copy(cnt_sh.at[pl.ds(r0 + i * C, C)], cb16)
            pltpu.sync_copy(cb16, acc_out.at[cid, pl.ds(r0 + i * C, C)])
            return carry

        pltpu.sync_copy(ones.at[pl.ds(C, C)], ones_v)
        lax.fori_loop(0, NRCHUNK, zstep, 0)
        pltpu.sync_copy(ones.at[pl.ds(0, C)], ones_v)
        lax.fori_loop(0, NSTEP, step, 0)
        plsc.subcore_barrier()
        lax.fori_loop(0, NRCHUNK, dstep, 0)

    return pl.kernel(
        body,
        out_type=jax.ShapeDtypeStruct((NC, ACC_ROWS, 16), jnp.float32),
        mesh=plsc.VectorSubcoreMesh(core_axis_name="c", subcore_axis_name="s"),
        scratch_types=[
            pltpu.VMEM((2, C), jnp.int32),
            pltpu.VMEM((C, 16), jnp.float32),
            pltpu.VMEM((C, 16), jnp.float32),
            pltpu.VMEM_SHARED((ACC_ROWS, 16), jnp.float32),
        ],
    )


def _tc_layer(acc, cnt, x_all, wl, bl, wr, gamma, beta, apply_bn: bool):
    RB = 1000
    NBU = N_USER // RB

    def body(acc_ref, cnt_ref, x_ref, wl_ref, bl_ref, wr_ref, g_ref, b_ref, o_ref):
        a = acc_ref[0]
        c = jnp.maximum(cnt_ref[0, :, 0:1], 1.0)
        mean = a / c
        h = lax.dot_general(mean, wl_ref[0], (((1,), (1,)), ((), ())),
                            preferred_element_type=jnp.float32)
        h = h + bl_ref[0]
        h = h + lax.dot_general(x_ref[...], wr_ref[0], (((1,), (1,)), ((), ())),
                                preferred_element_type=jnp.float32)
        if apply_bn:
            h = h * g_ref[0] + b_ref[0]
            h = jnp.where(h >= 0, h, h * SLOPE)
        o_ref[...] = h

    grid = (N_ALL // RB,)
    typ = lambda b: b // NBU
    in_specs = [
        pl.BlockSpec((1, RB, D), lambda b: (1 - b // NBU, b % NBU, 0)),
        pl.BlockSpec((1, RB, 16), lambda b: (1 - b // NBU, b % NBU, 0)),
        pl.BlockSpec((RB, D), lambda b: (b, 0)),
        pl.BlockSpec((1, D, D), lambda b: (typ(b), 0, 0)),
        pl.BlockSpec((1, 1, D), lambda b: (typ(b), 0, 0)),
        pl.BlockSpec((1, D, D), lambda b: (typ(b), 0, 0)),
        pl.BlockSpec((1, 1, D), lambda b: (typ(b), 0, 0)),
        pl.BlockSpec((1, 1, D), lambda b: (typ(b), 0, 0)),
    ]
    return pl.pallas_call(
        body,
        grid=grid,
        in_specs=in_specs,
        out_specs=pl.BlockSpec((RB, D), lambda b: (b, 0)),
        out_shape=jax.ShapeDtypeStruct((N_ALL, D), jnp.float32),
    )(acc, cnt, x_all, wl, bl, wr, gamma, beta)


def kernel(x_user, x_item, edge_index_u2i, edge_index_i2u,
           Wl0_u2i, bl0_u2i, Wr0_u2i, Wl0_i2u, bl0_i2u, Wr0_i2u,
           Wl1_u2i, bl1_u2i, Wr1_u2i, Wl1_i2u, bl1_i2u, Wr1_i2u,
           gamma_user, beta_user, gamma_item, beta_item):
    su, du = edge_index_u2i[0], edge_index_u2i[1]
    si, di = edge_index_i2u[0], edge_index_i2u[1]

    x_all = jnp.concatenate([x_user, x_item], axis=0)
    npad = E_PAD - E
    k = jnp.arange(npad, dtype=jnp.int32)
    pad_src = k % 9973
    pad_dst = N_USER + (k % (ACC_ROWS - N_USER))
    srcs = jnp.stack([jnp.concatenate([su, pad_src]),
                      jnp.concatenate([si + N_USER, N_USER + pad_src])])
    dsts = jnp.stack([jnp.concatenate([du, pad_dst]),
                      jnp.concatenate([di, pad_dst])])
    sd = jnp.stack([srcs.reshape(NC, NS * NSTEP, C),
                    dsts.reshape(NC, NS * NSTEP, C)], axis=2)
    zeros = jnp.zeros((C, D), jnp.float32)
    ones = jnp.concatenate([jnp.ones((C, 16), jnp.float32),
                            jnp.zeros((C, 16), jnp.float32)])

    cnt = _make_cnt()(sd, ones)
    acc0 = _make_agg()(x_all, sd, zeros)

    wl0 = jnp.stack([Wl0_i2u, Wl0_u2i])
    bl0 = jnp.stack([bl0_i2u, bl0_u2i])[:, None, :]
    wr0 = jnp.stack([Wr0_i2u, Wr0_u2i])
    gam = jnp.stack([gamma_user, gamma_item])[:, None, :]
    bet = jnp.stack([beta_user, beta_item])[:, None, :]
    h_all = _tc_layer(acc0, cnt, x_all, wl0, bl0, wr0, gam, bet, True)

    acc1 = _make_agg()(h_all, sd, zeros)
    wl1 = jnp.stack([Wl1_i2u, Wl1_u2i])
    bl1 = jnp.stack([bl1_i2u, bl1_u2i])[:, None, :]
    wr1 = jnp.stack([Wr1_i2u, Wr1_u2i])
    out_all = _tc_layer(acc1, cnt, h_all, wl1, bl1, wr1, gam, bet, False)
    return out_all[:N_USER], out_all[N_USER:]
